# Initial kernel scaffold; baseline (speedup 1.0000x reference)
import jax
import jax.numpy as jnp
from jax.experimental import pallas as pl


def kernel(x, router_w, router_b, ln_scale, ln_bias, w1, b1, w2, b2):
    raise NotImplementedError("write your pallas kernel here")


# speedup vs baseline: 3.4790x; 3.4790x over previous
import functools
import math

import jax
import jax.numpy as jnp
from jax.experimental import pallas as pl
from jax.experimental.pallas import tpu as pltpu

_N, _D, _E, _H, _C, _TOPK = 2048, 768, 16, 128, 10, 4
_EPS = 1e-5
_BN = 256


def _moe_block_kernel(x_ref, rw_ref, rb_ref, lns_ref, lnb_ref, w1_ref, b1_ref,
                      w2_ref, b2_ref, weighted_ref, all_probs_ref, gate_ref):
    x = x_ref[...]

    gl = jnp.dot(x, rw_ref[...], preferred_element_type=jnp.float32)
    gl = gl + rb_ref[...]
    gl = gl - jnp.max(gl, axis=-1, keepdims=True)
    ge = jnp.exp(gl)
    gp = ge / jnp.sum(ge, axis=-1, keepdims=True)
    gate_ref[...] = gp

    eidx = jax.lax.broadcasted_iota(jnp.int32, (_BN, _E), 1)
    rank = jnp.zeros((_BN, _E), dtype=jnp.int32)
    for ep in range(_E):
        gpe = gp[:, ep:ep + 1]
        beats = (gpe > gp) | ((gpe == gp) & (ep < eidx))
        rank = rank + beats.astype(jnp.int32)
    wsel = jnp.where(rank < _TOPK, gp, 0.0)
    wsel = wsel / jnp.sum(wsel, axis=-1, keepdims=True)

    mu = jnp.mean(x, axis=-1, keepdims=True)
    xc = x - mu
    var = jnp.mean(xc * xc, axis=-1, keepdims=True)
    xn = xc * jax.lax.rsqrt(var + _EPS)

    inv_sqrt2 = 1.0 / math.sqrt(2.0)
    acc = jnp.zeros((_BN, _C), dtype=jnp.float32)
    for e in range(_E):
        he = xn * lns_ref[e][None, :] + lnb_ref[e][None, :]
        h1 = jnp.dot(he, w1_ref[e], preferred_element_type=jnp.float32)
        h1 = h1 + b1_ref[e][None, :]
        h1 = 0.5 * h1 * (1.0 + jax.lax.erf(h1 * inv_sqrt2))
        lo = jnp.dot(h1, w2_ref[e], preferred_element_type=jnp.float32)
        lo = lo + b2_ref[e][None, :]
        lo = lo - jnp.max(lo, axis=-1, keepdims=True)
        pe = jnp.exp(lo)
        pe = pe / jnp.sum(pe, axis=-1, keepdims=True)
        all_probs_ref[e] = pe
        acc = acc + wsel[:, e:e + 1] * pe
    weighted_ref[...] = acc


@jax.jit
def kernel(x, router_w, router_b, ln_scale, ln_bias, w1, b1, w2, b2):
    rb2 = router_b.reshape(1, _E)
    grid = (_N // _BN,)
    out_shapes = (
        jax.ShapeDtypeStruct((_N, _C), jnp.float32),
        jax.ShapeDtypeStruct((_E, _N, _C), jnp.float32),
        jax.ShapeDtypeStruct((_N, _E), jnp.float32),
    )
    in_specs = [
        pl.BlockSpec((_BN, _D), lambda i: (i, 0)),
        pl.BlockSpec((_D, _E), lambda i: (0, 0)),
        pl.BlockSpec((1, _E), lambda i: (0, 0)),
        pl.BlockSpec((_E, _D), lambda i: (0, 0)),
        pl.BlockSpec((_E, _D), lambda i: (0, 0)),
        pl.BlockSpec((_E, _D, _H), lambda i: (0, 0, 0)),
        pl.BlockSpec((_E, _H), lambda i: (0, 0)),
        pl.BlockSpec((_E, _H, _C), lambda i: (0, 0, 0)),
        pl.BlockSpec((_E, _C), lambda i: (0, 0)),
    ]
    out_specs = (
        pl.BlockSpec((_BN, _C), lambda i: (i, 0)),
        pl.BlockSpec((_E, _BN, _C), lambda i: (0, i, 0)),
        pl.BlockSpec((_BN, _E), lambda i: (i, 0)),
    )
    weighted, all_probs, gate_probs = pl.pallas_call(
        _moe_block_kernel,
        grid=grid,
        in_specs=in_specs,
        out_specs=out_specs,
        out_shape=out_shapes,
    )(x, router_w, rb2, ln_scale, ln_bias, w1, b1, w2, b2)
    return weighted, all_probs, gate_probs


# speedup vs baseline: 5.1008x; 1.4662x over previous
import math

import jax
import jax.numpy as jnp
from jax.experimental import pallas as pl
from jax.experimental.pallas import tpu as pltpu

_N, _D, _E, _H, _C, _TOPK = 2048, 768, 16, 128, 10, 4
_EH = _E * _H
_EC = _E * _C
_EPS = 1e-5
_BN = 256


def _moe_block_kernel(x_ref, rw_ref, rb_ref, lns_ref, lnb_ref, w1_ref, b1_ref,
                      w2_ref, b2_ref, weighted_ref, all_probs_ref, gate_ref,
                      w1s_ref, b1e_ref, w2bd_ref, b2c_ref):
    @pl.when(pl.program_id(0) == 0)
    def _fold():
        w2bd_ref[...] = jnp.zeros((_EH, _EC), jnp.bfloat16)
        for e in range(_E):
            s = lns_ref[e].reshape(_D, 1)
            w1s_ref[:, e * _H:(e + 1) * _H] = (s * w1_ref[e]).astype(jnp.bfloat16)
            tb = jnp.dot(lnb_ref[e].reshape(1, _D), w1_ref[e],
                         preferred_element_type=jnp.float32)
            b1e_ref[:, e * _H:(e + 1) * _H] = tb + b1_ref[e][None, :]
            w2bd_ref[e * _H:(e + 1) * _H, e * _C:(e + 1) * _C] = (
                w2_ref[e].astype(jnp.bfloat16))
            b2c_ref[:, e * _C:(e + 1) * _C] = b2_ref[e][None, :]

    x = x_ref[...]

    gl = jnp.dot(x, rw_ref[...], preferred_element_type=jnp.float32)
    gl = gl + rb_ref[...]
    gl = gl - jnp.max(gl, axis=-1, keepdims=True)
    ge = jnp.exp(gl)
    gp = ge / jnp.sum(ge, axis=-1, keepdims=True)
    gate_ref[...] = gp

    eidx = jax.lax.broadcasted_iota(jnp.int32, (_BN, _E), 1)
    rank = jnp.zeros((_BN, _E), dtype=jnp.int32)
    for ep in range(_E):
        gpe = gp[:, ep:ep + 1]
        beats = (gpe > gp) | ((gpe == gp) & (ep < eidx))
        rank = rank + beats.astype(jnp.int32)
    wsel = jnp.where(rank < _TOPK, gp, 0.0)
    wsel = wsel / jnp.sum(wsel, axis=-1, keepdims=True)

    mu = jnp.mean(x, axis=-1, keepdims=True)
    xc = x - mu
    var = jnp.mean(xc * xc, axis=-1, keepdims=True)
    xn = xc * jax.lax.rsqrt(var + _EPS)

    h = jnp.dot(xn.astype(jnp.bfloat16), w1s_ref[...],
                preferred_element_type=jnp.float32)
    h = h + b1e_ref[...]
    h = 0.5 * h * (1.0 + jax.lax.erf(h * (1.0 / math.sqrt(2.0))))
    lo = jnp.dot(h.astype(jnp.bfloat16), w2bd_ref[...],
                 preferred_element_type=jnp.float32)
    lo = lo + b2c_ref[...]

    m = jnp.max(lo, axis=-1, keepdims=True)
    p = jnp.exp(lo - m)
    seg_of_lane = jax.lax.broadcasted_iota(jnp.int32, (_EC, _E), 0) // _C
    ecol = jax.lax.broadcasted_iota(jnp.int32, (_EC, _E), 1)
    bt = (seg_of_lane == ecol).astype(jnp.float32)
    ssum = jnp.dot(p, bt, preferred_element_type=jnp.float32)
    sr = 1.0 / ssum
    b = bt.T
    probs = p * jnp.dot(sr, b, preferred_element_type=jnp.float32)
    for e in range(_E):
        all_probs_ref[e] = probs[:, e * _C:(e + 1) * _C]

    cw160 = jnp.dot(wsel * sr, b, preferred_element_type=jnp.float32)
    lane_c = jax.lax.broadcasted_iota(jnp.int32, (_EC, _C), 0) % _C
    ccol = jax.lax.broadcasted_iota(jnp.int32, (_EC, _C), 1)
    g = (lane_c == ccol).astype(jnp.float32)
    weighted_ref[...] = jnp.dot(cw160 * p, g,
                                preferred_element_type=jnp.float32)


@jax.jit
def kernel(x, router_w, router_b, ln_scale, ln_bias, w1, b1, w2, b2):
    rb2 = router_b.reshape(1, _E)
    grid = (_N // _BN,)
    out_shapes = (
        jax.ShapeDtypeStruct((_N, _C), jnp.float32),
        jax.ShapeDtypeStruct((_E, _N, _C), jnp.float32),
        jax.ShapeDtypeStruct((_N, _E), jnp.float32),
    )
    in_specs = [
        pl.BlockSpec((_BN, _D), lambda i: (i, 0)),
        pl.BlockSpec((_D, _E), lambda i: (0, 0)),
        pl.BlockSpec((1, _E), lambda i: (0, 0)),
        pl.BlockSpec((_E, _D), lambda i: (0, 0)),
        pl.BlockSpec((_E, _D), lambda i: (0, 0)),
        pl.BlockSpec((_E, _D, _H), lambda i: (0, 0, 0)),
        pl.BlockSpec((_E, _H), lambda i: (0, 0)),
        pl.BlockSpec((_E, _H, _C), lambda i: (0, 0, 0)),
        pl.BlockSpec((_E, _C), lambda i: (0, 0)),
    ]
    out_specs = (
        pl.BlockSpec((_BN, _C), lambda i: (i, 0)),
        pl.BlockSpec((_E, _BN, _C), lambda i: (0, i, 0)),
        pl.BlockSpec((_BN, _E), lambda i: (i, 0)),
    )
    scratch_shapes = [
        pltpu.VMEM((_D, _EH), jnp.bfloat16),
        pltpu.VMEM((1, _EH), jnp.float32),
        pltpu.VMEM((_EH, _EC), jnp.bfloat16),
        pltpu.VMEM((1, _EC), jnp.float32),
    ]
    weighted, all_probs, gate_probs = pl.pallas_call(
        _moe_block_kernel,
        grid=grid,
        in_specs=in_specs,
        out_specs=out_specs,
        out_shape=out_shapes,
        scratch_shapes=scratch_shapes,
    )(x, router_w, rb2, ln_scale, ln_bias, w1, b1, w2, b2)
    return weighted, all_probs, gate_probs


# speedup vs baseline: 5.5500x; 1.0881x over previous
import math

import jax
import jax.numpy as jnp
from jax.experimental import pallas as pl
from jax.experimental.pallas import tpu as pltpu

_N, _D, _E, _H, _C, _TOPK = 2048, 768, 16, 128, 10, 4
_EH = _E * _H
_EC = _E * _C
_EPS = 1e-5
_BN = 512


def _moe_block_kernel(x_ref, rw_ref, rb_ref, lns_ref, lnb_ref, w1_ref, b1_ref,
                      w2_ref, b2_ref, weighted_ref, all_probs_ref, gate_ref,
                      w1s_ref, b1e_ref, w2bd_ref, b2c_ref, bt_ref, b_ref, g_ref):
    @pl.when(pl.program_id(0) == 0)
    def _fold():
        w2bd_ref[...] = jnp.zeros((_EH, _EC), jnp.bfloat16)
        for e in range(_E):
            s = lns_ref[e].reshape(_D, 1)
            w1s_ref[:, e * _H:(e + 1) * _H] = (s * w1_ref[e]).astype(jnp.bfloat16)
            tb = jnp.dot(lnb_ref[e].reshape(1, _D), w1_ref[e],
                         preferred_element_type=jnp.float32)
            b1e_ref[:, e * _H:(e + 1) * _H] = tb + b1_ref[e][None, :]
            w2bd_ref[e * _H:(e + 1) * _H, e * _C:(e + 1) * _C] = (
                w2_ref[e].astype(jnp.bfloat16))
            b2c_ref[:, e * _C:(e + 1) * _C] = b2_ref[e][None, :]
        seg_of_lane = jax.lax.broadcasted_iota(jnp.int32, (_EC, _E), 0) // _C
        ecol = jax.lax.broadcasted_iota(jnp.int32, (_EC, _E), 1)
        bt_ref[...] = (seg_of_lane == ecol).astype(jnp.float32)
        seg_r = jax.lax.broadcasted_iota(jnp.int32, (_E, _EC), 0)
        lane_r = jax.lax.broadcasted_iota(jnp.int32, (_E, _EC), 1) // _C
        b_ref[...] = (seg_r == lane_r).astype(jnp.float32)
        lane_c = jax.lax.broadcasted_iota(jnp.int32, (_EC, _C), 0) % _C
        ccol = jax.lax.broadcasted_iota(jnp.int32, (_EC, _C), 1)
        g_ref[...] = (lane_c == ccol).astype(jnp.float32)

    x = x_ref[...]

    gl = jnp.dot(x, rw_ref[...], preferred_element_type=jnp.float32)
    gl = gl + rb_ref[...]
    gl = gl - jnp.max(gl, axis=-1, keepdims=True)
    ge = jnp.exp(gl)
    gp = ge / jnp.sum(ge, axis=-1, keepdims=True)
    gate_ref[...] = gp

    gpt = gp.T
    erow = jax.lax.broadcasted_iota(jnp.int32, (_E, _BN), 0)
    rankt = jnp.zeros((_E, _BN), dtype=jnp.int32)
    for ep in range(_E):
        row = gpt[ep:ep + 1, :]
        beats = (row > gpt) | ((row == gpt) & (ep < erow))
        rankt = rankt + beats.astype(jnp.int32)
    wsel = jnp.where(rankt < _TOPK, gpt, 0.0).T
    wsel = wsel / jnp.sum(wsel, axis=-1, keepdims=True)

    mu = jnp.mean(x, axis=-1, keepdims=True)
    xc = x - mu
    var = jnp.mean(xc * xc, axis=-1, keepdims=True)
    xn = xc * jax.lax.rsqrt(var + _EPS)

    h = jnp.dot(xn.astype(jnp.bfloat16), w1s_ref[...],
                preferred_element_type=jnp.float32)
    h = h + b1e_ref[...]
    h = 0.5 * h * (1.0 + jax.lax.erf(h * (1.0 / math.sqrt(2.0))))
    lo = jnp.dot(h.astype(jnp.bfloat16), w2bd_ref[...],
                 preferred_element_type=jnp.float32)
    lo = lo + b2c_ref[...]

    m = jnp.max(lo, axis=-1, keepdims=True)
    p = jnp.exp(lo - m)
    ssum = jnp.dot(p, bt_ref[...], preferred_element_type=jnp.float32)
    sr = 1.0 / ssum
    probs = p * jnp.dot(sr, b_ref[...], preferred_element_type=jnp.float32)
    for e in range(_E):
        all_probs_ref[e] = probs[:, e * _C:(e + 1) * _C]

    cw160 = jnp.dot(wsel * sr, b_ref[...],
                    preferred_element_type=jnp.float32)
    weighted_ref[...] = jnp.dot(cw160 * p, g_ref[...],
                                preferred_element_type=jnp.float32)


@jax.jit
def kernel(x, router_w, router_b, ln_scale, ln_bias, w1, b1, w2, b2):
    rb2 = router_b.reshape(1, _E)
    grid = (_N // _BN,)
    out_shapes = (
        jax.ShapeDtypeStruct((_N, _C), jnp.float32),
        jax.ShapeDtypeStruct((_E, _N, _C), jnp.float32),
        jax.ShapeDtypeStruct((_N, _E), jnp.float32),
    )
    in_specs = [
        pl.BlockSpec((_BN, _D), lambda i: (i, 0)),
        pl.BlockSpec((_D, _E), lambda i: (0, 0)),
        pl.BlockSpec((1, _E), lambda i: (0, 0)),
        pl.BlockSpec((_E, _D), lambda i: (0, 0)),
        pl.BlockSpec((_E, _D), lambda i: (0, 0)),
        pl.BlockSpec((_E, _D, _H), lambda i: (0, 0, 0)),
        pl.BlockSpec((_E, _H), lambda i: (0, 0)),
        pl.BlockSpec((_E, _H, _C), lambda i: (0, 0, 0)),
        pl.BlockSpec((_E, _C), lambda i: (0, 0)),
    ]
    out_specs = (
        pl.BlockSpec((_BN, _C), lambda i: (i, 0)),
        pl.BlockSpec((_E, _BN, _C), lambda i: (0, i, 0)),
        pl.BlockSpec((_BN, _E), lambda i: (i, 0)),
    )
    scratch_shapes = [
        pltpu.VMEM((_D, _EH), jnp.bfloat16),
        pltpu.VMEM((1, _EH), jnp.float32),
        pltpu.VMEM((_EH, _EC), jnp.bfloat16),
        pltpu.VMEM((1, _EC), jnp.float32),
        pltpu.VMEM((_EC, _E), jnp.float32),
        pltpu.VMEM((_E, _EC), jnp.float32),
        pltpu.VMEM((_EC, _C), jnp.float32),
    ]
    weighted, all_probs, gate_probs = pl.pallas_call(
        _moe_block_kernel,
        grid=grid,
        in_specs=in_specs,
        out_specs=out_specs,
        out_shape=out_shapes,
        scratch_shapes=scratch_shapes,
    )(x, router_w, rb2, ln_scale, ln_bias, w1, b1, w2, b2)
    return weighted, all_probs, gate_probs
